# Initial kernel scaffold; baseline (speedup 1.0000x reference)
import jax
import jax.numpy as jnp
from jax.experimental import pallas as pl


def kernel(cls_heads, reg_heads, batch_anchors):
    raise NotImplementedError("write your pallas kernel here")


# speedup vs baseline: 12.8070x; 12.8070x over previous
import jax
import jax.numpy as jnp
from jax import lax
from jax.experimental import pallas as pl
from jax.experimental.pallas import tpu as pltpu

IMAGE_W = 512
IMAGE_H = 512
TOP_N = 1000
MIN_SCORE = 0.05
NMS_THR = 0.5
MAX_DET = 100

NPAD = 20480
ROWS = NPAD // 128
CAP = 1024
BROWS = 16
BLK = BROWS * 128
NBLK = ROWS // BROWS
RB = 256

_HI = jax.lax.Precision.HIGHEST


def _t(x):
    return jnp.transpose(x)


def _flat(x3):
    return jnp.concatenate([x3[:, r, :] for r in range(x3.shape[1])], axis=1)


def _score_class_kernel(cls_ref, s_ref, c_ref):
    x = cls_ref[0]
    mxk = jnp.max(x, axis=1, keepdims=True)
    idx = lax.broadcasted_iota(jnp.int32, x.shape, 1)
    amk = jnp.min(jnp.where(x == mxk, idx, 128), axis=1,
                  keepdims=True).astype(jnp.float32)
    s_ref[...] = jnp.reshape(_t(mxk), s_ref.shape)
    c_ref[...] = jnp.reshape(_t(amk), c_ref.shape)


def _compact_kernel(sfull_ref, sblk_ref, cblk_ref, reg_ref, anch_ref,
                    out_ref, sel_scr, pos_scr):
    k = pl.program_id(1)

    @pl.when(k == 0)
    def _select():
        s2d = sfull_ref[0]
        si = lax.bitcast_convert_type(s2d, jnp.int32)

        def _count_ge(v):
            return jnp.sum(jnp.where(si >= v, 1, 0), axis=1,
                           keepdims=True).sum(axis=0, keepdims=True)

        def _bs_body(_, st):
            lo, hi = st
            mid = lo + (hi - lo + 1) // 2
            ge = _count_ge(mid) >= TOP_N
            return (jnp.where(ge, mid, lo), jnp.where(ge, hi, mid - 1))

        lo0 = jnp.zeros((1, 1), jnp.int32)
        hi0 = jnp.full((1, 1), 0x3F800001, jnp.int32)
        tau, _ = lax.fori_loop(0, 31, _bs_body, (lo0, hi0))

        sel_gt = (si > tau).astype(jnp.float32)
        sel_eq = (si == tau).astype(jnp.float32)
        cnt_gt = jnp.sum(sel_gt, axis=1, keepdims=True).sum(
            axis=0, keepdims=True)
        need_eq = jnp.float32(TOP_N) - cnt_gt

        iu = lax.broadcasted_iota(jnp.int32, (128, 128), 0)
        ju = lax.broadcasted_iota(jnp.int32, (128, 128), 1)
        u128 = (iu <= ju).astype(jnp.float32)
        ir = lax.broadcasted_iota(jnp.int32, (ROWS, ROWS), 0)
        jr = lax.broadcasted_iota(jnp.int32, (ROWS, ROWS), 1)
        tstrict = (jr < ir).astype(jnp.float32)

        def _cum2d(m):
            row_incl = lax.dot_general(m, u128, (((1,), (0,)), ((), ())))
            rowtot = row_incl[:, 127:128]
            prefix = lax.dot_general(tstrict, rowtot,
                                     (((1,), (0,)), ((), ())))
            return row_incl + prefix

        eq_excl = _cum2d(sel_eq) - sel_eq
        sel = sel_gt + sel_eq * (eq_excl < need_eq).astype(jnp.float32)
        sel_scr[...] = sel
        pos_scr[...] = _cum2d(sel) - 1.0

    rsl = (pl.ds(k * BROWS, BROWS), slice(None))
    selb = sel_scr[rsl][None]
    posb = pos_scr[rsl][None]
    sb = sblk_ref[...]
    cb = cblk_ref[...]
    regb = reg_ref[0]
    anchb = anch_ref[0]

    sel_fr = _flat(selb)
    pos_fr = _flat(posb)
    rows11 = jnp.concatenate(
        [_flat(sb), _flat(cb), sel_fr, _flat(regb), _flat(anchb)],
        axis=0)

    slotc = lax.broadcasted_iota(jnp.int32, (CAP, BLK), 0).astype(jnp.float32)
    oht = (slotc == pos_fr).astype(jnp.float32) * sel_fr
    contrib = lax.dot_general(rows11, oht, (((1,), (1,)), ((), ())),
                              precision=_HI)

    @pl.when(k == 0)
    def _init():
        out_ref[...] = contrib[None]

    @pl.when(k > 0)
    def _acc():
        out_ref[...] = out_ref[...] + contrib[None]


def _finish_kernel(a_ref, os_ref, oc_ref, ob_ref):
    a = a_ref[0]
    s_row = a[0:1, :]
    real_row = a[2:3, :]

    iota_row = lax.broadcasted_iota(jnp.int32, (1, CAP), 1).astype(jnp.float32)
    s_er = jnp.where(real_row > 0.5, s_row, -1.0)
    i_er = jnp.where(real_row > 0.5, iota_row, 1e9)
    s_ec = _t(s_er)
    i_ec = _t(i_er)
    rank_blocks = []
    for rb in range(CAP // RB):
        s_b = s_ec[rb * RB:(rb + 1) * RB, :]
        i_b = i_ec[rb * RB:(rb + 1) * RB, :]
        before = ((s_er > s_b) |
                  ((s_er == s_b) & (i_er < i_b))).astype(jnp.float32)
        rank_blocks.append(jnp.sum(before, axis=1, keepdims=True))
    rank_row = _t(jnp.concatenate(rank_blocks, axis=0))

    iot_r = lax.broadcasted_iota(jnp.int32, (RB, CAP), 0).astype(jnp.float32)
    srt_b = []
    for rb in range(CAP // RB):
        perm = (rank_row == (iot_r + jnp.float32(rb * RB))).astype(
            jnp.float32)
        srt_b.append(lax.dot_general(a, perm, (((1,), (1,)), ((), ())),
                                     precision=_HI))
    srt = jnp.concatenate(srt_b, axis=1)

    sc = srt[0:1, :]
    cl = srt[1:2, :]
    rx = srt[3:4, :] * jnp.float32(0.1)
    ry = srt[4:5, :] * jnp.float32(0.1)
    rw = srt[5:6, :] * jnp.float32(0.2)
    rh = srt[6:7, :] * jnp.float32(0.2)
    ax1 = srt[7:8, :]
    ay1 = srt[8:9, :]
    ax2 = srt[9:10, :]
    ay2 = srt[10:11, :]

    awx = ax2 - ax1
    awy = ay2 - ay1
    acx = ax1 + 0.5 * awx
    acy = ay1 + 0.5 * awy
    pw = jnp.exp(rw) * awx
    ph = jnp.exp(rh) * awy
    pcx = rx * awx + acx
    pcy = ry * awy + acy
    b0 = jnp.maximum((pcx - 0.5 * pw).astype(jnp.int32), 0).astype(jnp.float32)
    b1 = jnp.maximum((pcy - 0.5 * ph).astype(jnp.int32), 0).astype(jnp.float32)
    b2 = jnp.minimum((pcx + 0.5 * pw).astype(jnp.int32),
                     IMAGE_W - 1).astype(jnp.float32)
    b3 = jnp.minimum((pcy + 0.5 * ph).astype(jnp.int32),
                     IMAGE_H - 1).astype(jnp.float32)

    rown = lax.broadcasted_iota(jnp.int32, (1, CAP), 1)
    valid_row = ((rown < TOP_N) & (sc > MIN_SCORE)).astype(jnp.float32)

    area = (b2 - b0) * (b3 - b1)
    b0c = _t(b0); b1c = _t(b1); b2c = _t(b2); b3c = _t(b3)
    areac = _t(area)
    strips = []
    for rb in range(CAP // RB):
        r0 = rb * RB
        sl = slice(r0, r0 + RB)
        xx1 = jnp.maximum(b0c[sl], b0)
        yy1 = jnp.maximum(b1c[sl], b1)
        xx2 = jnp.minimum(b2c[sl], b2)
        yy2 = jnp.minimum(b3c[sl], b3)
        w = jnp.maximum(xx2 - xx1, 0.0)
        h = jnp.maximum(yy2 - yy1, 0.0)
        inter = w * h
        union = areac[sl] + area - inter
        over = 2.0 * inter > jnp.maximum(union, 1e-9)
        icol = lax.broadcasted_iota(jnp.int32, (RB, CAP), 0) + r0
        jcol = lax.broadcasted_iota(jnp.int32, (RB, CAP), 1)
        strips.append((over & (icol < jcol)).astype(jnp.float32))
    supm = jnp.concatenate(strips, axis=0)

    def _nms_cond(st):
        return st[1]

    def _nms_body(st):
        keep, _ = st
        sup = lax.dot_general(keep, supm, (((1,), (0,)), ((), ())))
        newk = jnp.where(sup > 0.5, 0.0, valid_row)
        return (newk, jnp.any(newk != keep))

    keep_row, _ = lax.while_loop(_nms_cond, _nms_body,
                                 (valid_row, jnp.bool_(True)))

    cnt_b = []
    for rb in range(CAP // RB):
        r0 = rb * RB
        icol = lax.broadcasted_iota(jnp.int32, (RB, CAP), 0) + r0
        jcol = lax.broadcasted_iota(jnp.int32, (RB, CAP), 1)
        m = keep_row * (jcol < icol).astype(jnp.float32)
        cnt_b.append(jnp.sum(m, axis=1, keepdims=True))
    pos_row = _t(jnp.concatenate(cnt_b, axis=0))

    oslot = lax.broadcasted_iota(jnp.int32, (128, CAP), 0).astype(jnp.float32)
    oh = (oslot == pos_row).astype(jnp.float32) * keep_row
    outp = jnp.concatenate([sc, cl, b0, b1, b2, b3], axis=0)
    gath = lax.dot_general(outp, oh, (((1,), (1,)), ((), ())),
                           precision=_HI)
    filled = _t(jnp.sum(oh, axis=1, keepdims=True))
    final = jnp.where(filled > 0.5, gath, -1.0)

    os_ref[...] = jnp.reshape(final[0:1, :MAX_DET], (1, 1, MAX_DET))
    oc_ref[...] = jnp.reshape(final[1:2, :MAX_DET], (1, 1, MAX_DET))
    ob_ref[...] = jnp.reshape(_t(final[2:6, :MAX_DET]), (1, MAX_DET, 4))


@jax.jit
def kernel(cls_heads, reg_heads, batch_anchors):
    B, N, C = cls_heads.shape
    nblk_a = 10
    blk_a = N // nblk_a

    scores4, classes4 = pl.pallas_call(
        _score_class_kernel,
        grid=(B, nblk_a),
        in_specs=[pl.BlockSpec((1, blk_a, C), lambda b, k: (b, k, 0))],
        out_specs=[pl.BlockSpec((1, 1, 1, blk_a), lambda b, k: (b, k, 0, 0)),
                   pl.BlockSpec((1, 1, 1, blk_a), lambda b, k: (b, k, 0, 0))],
        out_shape=[jax.ShapeDtypeStruct((B, nblk_a, 1, blk_a), jnp.float32),
                   jax.ShapeDtypeStruct((B, nblk_a, 1, blk_a), jnp.float32)],
    )(cls_heads)
    scores = jnp.reshape(scores4, (B, N))
    classes = jnp.reshape(classes4, (B, N))

    pad = NPAD - N
    s2d = jnp.reshape(jnp.pad(scores, ((0, 0), (0, pad)),
                              constant_values=-1.0), (B, ROWS, 128))
    c2d = jnp.reshape(jnp.pad(classes, ((0, 0), (0, pad))), (B, ROWS, 128))
    regT = jnp.reshape(jnp.transpose(
        jnp.pad(reg_heads, ((0, 0), (0, pad), (0, 0))), (0, 2, 1)),
        (B, 4, ROWS, 128))
    anchT = jnp.reshape(jnp.transpose(
        jnp.pad(batch_anchors, ((0, 0), (0, pad), (0, 0))), (0, 2, 1)),
        (B, 4, ROWS, 128))

    compact = pl.pallas_call(
        _compact_kernel,
        grid=(B, NBLK),
        in_specs=[pl.BlockSpec((1, ROWS, 128), lambda b, k: (b, 0, 0)),
                  pl.BlockSpec((1, BROWS, 128), lambda b, k: (b, k, 0)),
                  pl.BlockSpec((1, BROWS, 128), lambda b, k: (b, k, 0)),
                  pl.BlockSpec((1, 4, BROWS, 128), lambda b, k: (b, 0, k, 0)),
                  pl.BlockSpec((1, 4, BROWS, 128), lambda b, k: (b, 0, k, 0))],
        out_specs=pl.BlockSpec((1, 11, CAP), lambda b, k: (b, 0, 0)),
        out_shape=jax.ShapeDtypeStruct((B, 11, CAP), jnp.float32),
        scratch_shapes=[pltpu.VMEM((ROWS, 128), jnp.float32),
                        pltpu.VMEM((ROWS, 128), jnp.float32)],
    )(s2d, s2d, c2d, regT, anchT)

    out_s, out_c, out_b = pl.pallas_call(
        _finish_kernel,
        grid=(B,),
        in_specs=[pl.BlockSpec((1, 11, CAP), lambda b: (b, 0, 0))],
        out_specs=[pl.BlockSpec((1, 1, MAX_DET), lambda b: (b, 0, 0)),
                   pl.BlockSpec((1, 1, MAX_DET), lambda b: (b, 0, 0)),
                   pl.BlockSpec((1, MAX_DET, 4), lambda b: (b, 0, 0))],
        out_shape=[jax.ShapeDtypeStruct((B, 1, MAX_DET), jnp.float32),
                   jax.ShapeDtypeStruct((B, 1, MAX_DET), jnp.float32),
                   jax.ShapeDtypeStruct((B, MAX_DET, 4), jnp.float32)],
    )(compact)

    return (jnp.reshape(out_s, (B, MAX_DET)),
            jnp.reshape(out_c, (B, MAX_DET)),
            out_b)


# speedup vs baseline: 25.8873x; 2.0213x over previous
import jax
import jax.numpy as jnp
from jax import lax
from jax.experimental import pallas as pl
from jax.experimental.pallas import tpu as pltpu

IMAGE_W = 512
IMAGE_H = 512
TOP_N = 1000
MIN_SCORE = 0.05
NMS_THR = 0.5
MAX_DET = 100

NPAD = 20480
ROWS = NPAD // 128
CAP = 1024
BROWS = 16
BLK = BROWS * 128
NBLK = ROWS // BROWS
RB = 256

_HI = jax.lax.Precision.HIGHEST


def _t(x):
    return jnp.transpose(x)


def _flat(x3):
    return jnp.concatenate([x3[:, r, :] for r in range(x3.shape[1])], axis=1)


def _score_class_kernel(cls_ref, s_ref, c_ref):
    x = cls_ref[0]
    mxk = jnp.max(x, axis=1, keepdims=True)
    idx = lax.broadcasted_iota(jnp.int32, x.shape, 1)
    amk = jnp.min(jnp.where(x == mxk, idx, 128), axis=1,
                  keepdims=True).astype(jnp.float32)
    s_ref[...] = jnp.reshape(_t(mxk), s_ref.shape)
    c_ref[...] = jnp.reshape(_t(amk), c_ref.shape)


def _compact_kernel(sfull_ref, sblk_ref, cblk_ref, reg_ref, anch_ref,
                    out_ref, sel_scr, pos_scr):
    k = pl.program_id(1)

    @pl.when(k == 0)
    def _select():
        s2d = sfull_ref[0]
        si = lax.bitcast_convert_type(s2d, jnp.int32)

        def _count_ge(v):
            return jnp.sum(jnp.where(si >= v, 1, 0), axis=1,
                           keepdims=True).sum(axis=0, keepdims=True)

        def _bs_body(_, st):
            lo, hi = st
            mid = lo + (hi - lo + 1) // 2
            ge = _count_ge(mid) >= TOP_N
            return (jnp.where(ge, mid, lo), jnp.where(ge, hi, mid - 1))

        lo0 = jnp.zeros((1, 1), jnp.int32)
        hi0 = jnp.full((1, 1), 0x3F800001, jnp.int32)
        tau, _ = lax.fori_loop(0, 31, _bs_body, (lo0, hi0))

        sel_gt = (si > tau).astype(jnp.float32)
        sel_eq = (si == tau).astype(jnp.float32)
        cnt_gt = jnp.sum(sel_gt, axis=1, keepdims=True).sum(
            axis=0, keepdims=True)
        need_eq = jnp.float32(TOP_N) - cnt_gt

        iu = lax.broadcasted_iota(jnp.int32, (128, 128), 0)
        ju = lax.broadcasted_iota(jnp.int32, (128, 128), 1)
        u128 = (iu <= ju).astype(jnp.float32)
        ir = lax.broadcasted_iota(jnp.int32, (ROWS, ROWS), 0)
        jr = lax.broadcasted_iota(jnp.int32, (ROWS, ROWS), 1)
        tstrict = (jr < ir).astype(jnp.float32)

        def _cum2d(m):
            row_incl = lax.dot_general(m, u128, (((1,), (0,)), ((), ())))
            rowtot = row_incl[:, 127:128]
            prefix = lax.dot_general(tstrict, rowtot,
                                     (((1,), (0,)), ((), ())))
            return row_incl + prefix

        eq_excl = _cum2d(sel_eq) - sel_eq
        sel = sel_gt + sel_eq * (eq_excl < need_eq).astype(jnp.float32)
        sel_scr[...] = sel
        pos_scr[...] = _cum2d(sel) - 1.0

    rsl = (pl.ds(k * BROWS, BROWS), slice(None))
    selb = sel_scr[rsl][None]
    posb = pos_scr[rsl][None]
    sb = sblk_ref[...]
    cb = cblk_ref[...]
    regb = reg_ref[0]
    anchb = anch_ref[0]

    data3 = jnp.concatenate([sb, cb, selb, regb, anchb],
                            axis=0)
    rows11 = _flat(data3)
    sel_fr = rows11[2:3, :]
    pos_fr = _flat(posb)

    pos_hi = jnp.floor(pos_fr * jnp.float32(1.0 / 128.0))
    pos_lo = pos_fr - jnp.float32(128.0) * pos_hi
    hic = lax.broadcasted_iota(jnp.int32, (CAP // 128, BLK), 0).astype(
        jnp.float32)
    loc = lax.broadcasted_iota(jnp.int32, (128, BLK), 0).astype(jnp.float32)
    amask = (hic == pos_hi).astype(jnp.float32) * sel_fr
    bmask = (loc == pos_lo).astype(jnp.float32)
    mrows = jnp.concatenate(
        [rows11 * amask[hi:hi + 1, :] for hi in range(CAP // 128)],
        axis=0)
    big = lax.dot_general(mrows, bmask, (((1,), (1,)), ((), ())),
                          precision=_HI)
    contrib = jnp.concatenate(
        [big[11 * hi:11 * (hi + 1), :] for hi in range(CAP // 128)],
        axis=1)

    @pl.when(k == 0)
    def _init():
        out_ref[...] = contrib[None]

    @pl.when(k > 0)
    def _acc():
        out_ref[...] = out_ref[...] + contrib[None]


def _finish_kernel(a_ref, os_ref, oc_ref, ob_ref):
    a = a_ref[0]
    s_row = a[0:1, :]
    real_row = a[2:3, :]

    iota_row = lax.broadcasted_iota(jnp.int32, (1, CAP), 1).astype(jnp.float32)
    s_er = jnp.where(real_row > 0.5, s_row, -1.0)
    i_er = jnp.where(real_row > 0.5, iota_row, 1e9)
    s_ec = _t(s_er)
    i_ec = _t(i_er)
    rank_blocks = []
    for rb in range(CAP // RB):
        s_b = s_ec[rb * RB:(rb + 1) * RB, :]
        i_b = i_ec[rb * RB:(rb + 1) * RB, :]
        before = ((s_er > s_b) |
                  ((s_er == s_b) & (i_er < i_b))).astype(jnp.float32)
        rank_blocks.append(jnp.sum(before, axis=1, keepdims=True))
    rank_row = _t(jnp.concatenate(rank_blocks, axis=0))

    iot_r = lax.broadcasted_iota(jnp.int32, (RB, CAP), 0).astype(jnp.float32)
    srt_b = []
    for rb in range(CAP // RB):
        perm = (rank_row == (iot_r + jnp.float32(rb * RB))).astype(
            jnp.float32)
        srt_b.append(lax.dot_general(a, perm, (((1,), (1,)), ((), ())),
                                     precision=_HI))
    srt = jnp.concatenate(srt_b, axis=1)

    sc = srt[0:1, :]
    cl = srt[1:2, :]
    rx = srt[3:4, :] * jnp.float32(0.1)
    ry = srt[4:5, :] * jnp.float32(0.1)
    rw = srt[5:6, :] * jnp.float32(0.2)
    rh = srt[6:7, :] * jnp.float32(0.2)
    ax1 = srt[7:8, :]
    ay1 = srt[8:9, :]
    ax2 = srt[9:10, :]
    ay2 = srt[10:11, :]

    awx = ax2 - ax1
    awy = ay2 - ay1
    acx = ax1 + 0.5 * awx
    acy = ay1 + 0.5 * awy
    pw = jnp.exp(rw) * awx
    ph = jnp.exp(rh) * awy
    pcx = rx * awx + acx
    pcy = ry * awy + acy
    b0 = jnp.maximum((pcx - 0.5 * pw).astype(jnp.int32), 0).astype(jnp.float32)
    b1 = jnp.maximum((pcy - 0.5 * ph).astype(jnp.int32), 0).astype(jnp.float32)
    b2 = jnp.minimum((pcx + 0.5 * pw).astype(jnp.int32),
                     IMAGE_W - 1).astype(jnp.float32)
    b3 = jnp.minimum((pcy + 0.5 * ph).astype(jnp.int32),
                     IMAGE_H - 1).astype(jnp.float32)

    rown = lax.broadcasted_iota(jnp.int32, (1, CAP), 1)
    valid_row = ((rown < TOP_N) & (sc > MIN_SCORE)).astype(jnp.float32)

    area = (b2 - b0) * (b3 - b1)
    b0c = _t(b0); b1c = _t(b1); b2c = _t(b2); b3c = _t(b3)
    areac = _t(area)
    strips = []
    for rb in range(CAP // RB):
        r0 = rb * RB
        sl = slice(r0, r0 + RB)
        xx1 = jnp.maximum(b0c[sl], b0)
        yy1 = jnp.maximum(b1c[sl], b1)
        xx2 = jnp.minimum(b2c[sl], b2)
        yy2 = jnp.minimum(b3c[sl], b3)
        w = jnp.maximum(xx2 - xx1, 0.0)
        h = jnp.maximum(yy2 - yy1, 0.0)
        inter = w * h
        union = areac[sl] + area - inter
        over = 2.0 * inter > jnp.maximum(union, 1e-9)
        icol = lax.broadcasted_iota(jnp.int32, (RB, CAP), 0) + r0
        jcol = lax.broadcasted_iota(jnp.int32, (RB, CAP), 1)
        strips.append((over & (icol < jcol)).astype(jnp.float32))
    supm = jnp.concatenate(strips, axis=0)

    def _nms_cond(st):
        return st[1]

    def _nms_body(st):
        keep, _ = st
        sup = lax.dot_general(keep, supm, (((1,), (0,)), ((), ())))
        newk = jnp.where(sup > 0.5, 0.0, valid_row)
        return (newk, jnp.any(newk != keep))

    keep_row, _ = lax.while_loop(_nms_cond, _nms_body,
                                 (valid_row, jnp.bool_(True)))

    cnt_b = []
    for rb in range(CAP // RB):
        r0 = rb * RB
        icol = lax.broadcasted_iota(jnp.int32, (RB, CAP), 0) + r0
        jcol = lax.broadcasted_iota(jnp.int32, (RB, CAP), 1)
        m = keep_row * (jcol < icol).astype(jnp.float32)
        cnt_b.append(jnp.sum(m, axis=1, keepdims=True))
    pos_row = _t(jnp.concatenate(cnt_b, axis=0))

    oslot = lax.broadcasted_iota(jnp.int32, (128, CAP), 0).astype(jnp.float32)
    oh = (oslot == pos_row).astype(jnp.float32) * keep_row
    outp = jnp.concatenate([sc, cl, b0, b1, b2, b3], axis=0)
    gath = lax.dot_general(outp, oh, (((1,), (1,)), ((), ())),
                           precision=_HI)
    filled = _t(jnp.sum(oh, axis=1, keepdims=True))
    final = jnp.where(filled > 0.5, gath, -1.0)

    os_ref[...] = jnp.reshape(final[0:1, :MAX_DET], (1, 1, MAX_DET))
    oc_ref[...] = jnp.reshape(final[1:2, :MAX_DET], (1, 1, MAX_DET))
    ob_ref[...] = jnp.reshape(_t(final[2:6, :MAX_DET]), (1, MAX_DET, 4))


@jax.jit
def kernel(cls_heads, reg_heads, batch_anchors):
    B, N, C = cls_heads.shape
    nblk_a = 10
    blk_a = N // nblk_a

    scores4, classes4 = pl.pallas_call(
        _score_class_kernel,
        grid=(B, nblk_a),
        in_specs=[pl.BlockSpec((1, blk_a, C), lambda b, k: (b, k, 0))],
        out_specs=[pl.BlockSpec((1, 1, 1, blk_a), lambda b, k: (b, k, 0, 0)),
                   pl.BlockSpec((1, 1, 1, blk_a), lambda b, k: (b, k, 0, 0))],
        out_shape=[jax.ShapeDtypeStruct((B, nblk_a, 1, blk_a), jnp.float32),
                   jax.ShapeDtypeStruct((B, nblk_a, 1, blk_a), jnp.float32)],
    )(cls_heads)
    scores = jnp.reshape(scores4, (B, N))
    classes = jnp.reshape(classes4, (B, N))

    pad = NPAD - N
    s2d = jnp.reshape(jnp.pad(scores, ((0, 0), (0, pad)),
                              constant_values=-1.0), (B, ROWS, 128))
    c2d = jnp.reshape(jnp.pad(classes, ((0, 0), (0, pad))), (B, ROWS, 128))
    regT = jnp.reshape(jnp.transpose(
        jnp.pad(reg_heads, ((0, 0), (0, pad), (0, 0))), (0, 2, 1)),
        (B, 4, ROWS, 128))
    anchT = jnp.reshape(jnp.transpose(
        jnp.pad(batch_anchors, ((0, 0), (0, pad), (0, 0))), (0, 2, 1)),
        (B, 4, ROWS, 128))

    compact = pl.pallas_call(
        _compact_kernel,
        grid=(B, NBLK),
        in_specs=[pl.BlockSpec((1, ROWS, 128), lambda b, k: (b, 0, 0)),
                  pl.BlockSpec((1, BROWS, 128), lambda b, k: (b, k, 0)),
                  pl.BlockSpec((1, BROWS, 128), lambda b, k: (b, k, 0)),
                  pl.BlockSpec((1, 4, BROWS, 128), lambda b, k: (b, 0, k, 0)),
                  pl.BlockSpec((1, 4, BROWS, 128), lambda b, k: (b, 0, k, 0))],
        out_specs=pl.BlockSpec((1, 11, CAP), lambda b, k: (b, 0, 0)),
        out_shape=jax.ShapeDtypeStruct((B, 11, CAP), jnp.float32),
        scratch_shapes=[pltpu.VMEM((ROWS, 128), jnp.float32),
                        pltpu.VMEM((ROWS, 128), jnp.float32)],
    )(s2d, s2d, c2d, regT, anchT)

    out_s, out_c, out_b = pl.pallas_call(
        _finish_kernel,
        grid=(B,),
        in_specs=[pl.BlockSpec((1, 11, CAP), lambda b: (b, 0, 0))],
        out_specs=[pl.BlockSpec((1, 1, MAX_DET), lambda b: (b, 0, 0)),
                   pl.BlockSpec((1, 1, MAX_DET), lambda b: (b, 0, 0)),
                   pl.BlockSpec((1, MAX_DET, 4), lambda b: (b, 0, 0))],
        out_shape=[jax.ShapeDtypeStruct((B, 1, MAX_DET), jnp.float32),
                   jax.ShapeDtypeStruct((B, 1, MAX_DET), jnp.float32),
                   jax.ShapeDtypeStruct((B, MAX_DET, 4), jnp.float32)],
    )(compact)

    return (jnp.reshape(out_s, (B, MAX_DET)),
            jnp.reshape(out_c, (B, MAX_DET)),
            out_b)


# speedup vs baseline: 27.4955x; 1.0621x over previous
import jax
import jax.numpy as jnp
from jax import lax
from jax.experimental import pallas as pl
from jax.experimental.pallas import tpu as pltpu

IMAGE_W = 512
IMAGE_H = 512
TOP_N = 1000
MIN_SCORE = 0.05
NMS_THR = 0.5
MAX_DET = 100

NPAD = 20480
ROWS = NPAD // 128
CAP = 1024
BROWS = 16
BLK = BROWS * 128
NBLK = ROWS // BROWS
RB = 256

_HI = jax.lax.Precision.HIGHEST


def _t(x):
    return jnp.transpose(x)


def _flat(x3):
    return jnp.concatenate([x3[:, r, :] for r in range(x3.shape[1])], axis=1)


def _score_class_kernel(cls_ref, s_ref, c_ref):
    x = cls_ref[0]
    C = x.shape[1]
    mxk = jnp.max(x, axis=1, keepdims=True)
    mask = (x == mxk).astype(jnp.float32)
    iu = lax.broadcasted_iota(jnp.int32, (C, C), 0)
    ju = lax.broadcasted_iota(jnp.int32, (C, C), 1)
    tri = (iu < ju).astype(jnp.float32)
    cnt = lax.dot_general(mask, tri, (((1,), (0,)), ((), ())))
    fm = mask * (cnt == 0).astype(jnp.float32)
    idxc = lax.broadcasted_iota(jnp.int32, (C, 1), 0).astype(jnp.float32)
    amk = lax.dot_general(fm, idxc, (((1,), (0,)), ((), ())))
    s_ref[...] = jnp.reshape(_t(mxk), s_ref.shape)
    c_ref[...] = jnp.reshape(_t(amk), c_ref.shape)


def _compact_kernel(sfull_ref, sblk_ref, cblk_ref, reg_ref, anch_ref,
                    out_ref, sel_scr, pos_scr):
    k = pl.program_id(1)

    @pl.when(k == 0)
    def _select():
        s2d = sfull_ref[0]
        si = lax.bitcast_convert_type(s2d, jnp.int32)

        def _count_ge(v):
            return jnp.sum(jnp.where(si >= v, 1, 0), axis=1,
                           keepdims=True).sum(axis=0, keepdims=True)

        def _bs_body(_, st):
            lo, hi = st
            mid = lo + (hi - lo + 1) // 2
            ge = _count_ge(mid) >= TOP_N
            return (jnp.where(ge, mid, lo), jnp.where(ge, hi, mid - 1))

        lo0 = jnp.zeros((1, 1), jnp.int32)
        hi0 = jnp.full((1, 1), 0x3F800001, jnp.int32)
        tau, _ = lax.fori_loop(0, 31, _bs_body, (lo0, hi0))

        sel_gt = (si > tau).astype(jnp.float32)
        sel_eq = (si == tau).astype(jnp.float32)
        cnt_gt = jnp.sum(sel_gt, axis=1, keepdims=True).sum(
            axis=0, keepdims=True)
        need_eq = jnp.float32(TOP_N) - cnt_gt

        iu = lax.broadcasted_iota(jnp.int32, (128, 128), 0)
        ju = lax.broadcasted_iota(jnp.int32, (128, 128), 1)
        u128 = (iu <= ju).astype(jnp.float32)
        ir = lax.broadcasted_iota(jnp.int32, (ROWS, ROWS), 0)
        jr = lax.broadcasted_iota(jnp.int32, (ROWS, ROWS), 1)
        tstrict = (jr < ir).astype(jnp.float32)

        def _cum2d(m):
            row_incl = lax.dot_general(m, u128, (((1,), (0,)), ((), ())))
            rowtot = row_incl[:, 127:128]
            prefix = lax.dot_general(tstrict, rowtot,
                                     (((1,), (0,)), ((), ())))
            return row_incl + prefix

        eq_excl = _cum2d(sel_eq) - sel_eq
        sel = sel_gt + sel_eq * (eq_excl < need_eq).astype(jnp.float32)
        sel_scr[...] = sel
        pos_scr[...] = _cum2d(sel) - 1.0

    rsl = (pl.ds(k * BROWS, BROWS), slice(None))
    selb = sel_scr[rsl][None]
    posb = pos_scr[rsl][None]
    sb = sblk_ref[...]
    cb = cblk_ref[...]
    regb = reg_ref[0]
    anchb = anch_ref[0]

    data3 = jnp.concatenate([sb, cb, selb, regb, anchb],
                            axis=0)
    rows11 = _flat(data3)
    sel_fr = rows11[2:3, :]
    pos_fr = _flat(posb)

    pos_hi = jnp.floor(pos_fr * jnp.float32(1.0 / 128.0))
    pos_lo = pos_fr - jnp.float32(128.0) * pos_hi
    hic = lax.broadcasted_iota(jnp.int32, (CAP // 128, BLK), 0).astype(
        jnp.float32)
    loc = lax.broadcasted_iota(jnp.int32, (128, BLK), 0).astype(jnp.float32)
    amask = (hic == pos_hi).astype(jnp.float32) * sel_fr
    bmask = (loc == pos_lo).astype(jnp.float32)
    mrows = jnp.concatenate(
        [rows11 * amask[hi:hi + 1, :] for hi in range(CAP // 128)],
        axis=0)
    big = lax.dot_general(mrows, bmask, (((1,), (1,)), ((), ())),
                          precision=_HI)
    contrib = jnp.concatenate(
        [big[11 * hi:11 * (hi + 1), :] for hi in range(CAP // 128)],
        axis=1)

    @pl.when(k == 0)
    def _init():
        out_ref[...] = contrib[None]

    @pl.when(k > 0)
    def _acc():
        out_ref[...] = out_ref[...] + contrib[None]


def _finish_kernel(a_ref, os_ref, oc_ref, ob_ref):
    a = a_ref[0]
    s_row = a[0:1, :]
    real_row = a[2:3, :]

    iota_row = lax.broadcasted_iota(jnp.int32, (1, CAP), 1).astype(jnp.float32)
    s_er = jnp.where(real_row > 0.5, s_row, -1.0)
    i_er = jnp.where(real_row > 0.5, iota_row, 1e9)
    s_ec = _t(s_er)
    i_ec = _t(i_er)
    rank_blocks = []
    for rb in range(CAP // RB):
        s_b = s_ec[rb * RB:(rb + 1) * RB, :]
        i_b = i_ec[rb * RB:(rb + 1) * RB, :]
        before = ((s_er > s_b) |
                  ((s_er == s_b) & (i_er < i_b))).astype(jnp.float32)
        rank_blocks.append(jnp.sum(before, axis=1, keepdims=True))
    rank_row = _t(jnp.concatenate(rank_blocks, axis=0))

    iot_r = lax.broadcasted_iota(jnp.int32, (RB, CAP), 0).astype(jnp.float32)
    srt_b = []
    for rb in range(CAP // RB):
        perm = (rank_row == (iot_r + jnp.float32(rb * RB))).astype(
            jnp.float32)
        srt_b.append(lax.dot_general(a, perm, (((1,), (1,)), ((), ())),
                                     precision=_HI))
    srt = jnp.concatenate(srt_b, axis=1)

    sc = srt[0:1, :]
    cl = srt[1:2, :]
    rx = srt[3:4, :] * jnp.float32(0.1)
    ry = srt[4:5, :] * jnp.float32(0.1)
    rw = srt[5:6, :] * jnp.float32(0.2)
    rh = srt[6:7, :] * jnp.float32(0.2)
    ax1 = srt[7:8, :]
    ay1 = srt[8:9, :]
    ax2 = srt[9:10, :]
    ay2 = srt[10:11, :]

    awx = ax2 - ax1
    awy = ay2 - ay1
    acx = ax1 + 0.5 * awx
    acy = ay1 + 0.5 * awy
    pw = jnp.exp(rw) * awx
    ph = jnp.exp(rh) * awy
    pcx = rx * awx + acx
    pcy = ry * awy + acy
    b0 = jnp.maximum((pcx - 0.5 * pw).astype(jnp.int32), 0).astype(jnp.float32)
    b1 = jnp.maximum((pcy - 0.5 * ph).astype(jnp.int32), 0).astype(jnp.float32)
    b2 = jnp.minimum((pcx + 0.5 * pw).astype(jnp.int32),
                     IMAGE_W - 1).astype(jnp.float32)
    b3 = jnp.minimum((pcy + 0.5 * ph).astype(jnp.int32),
                     IMAGE_H - 1).astype(jnp.float32)

    rown = lax.broadcasted_iota(jnp.int32, (1, CAP), 1)
    valid_row = ((rown < TOP_N) & (sc > MIN_SCORE)).astype(jnp.float32)

    area = (b2 - b0) * (b3 - b1)
    b0c = _t(b0); b1c = _t(b1); b2c = _t(b2); b3c = _t(b3)
    areac = _t(area)
    strips = []
    for rb in range(CAP // RB):
        r0 = rb * RB
        sl = slice(r0, r0 + RB)
        xx1 = jnp.maximum(b0c[sl], b0)
        yy1 = jnp.maximum(b1c[sl], b1)
        xx2 = jnp.minimum(b2c[sl], b2)
        yy2 = jnp.minimum(b3c[sl], b3)
        w = jnp.maximum(xx2 - xx1, 0.0)
        h = jnp.maximum(yy2 - yy1, 0.0)
        inter = w * h
        union = areac[sl] + area - inter
        over = 2.0 * inter > jnp.maximum(union, 1e-9)
        icol = lax.broadcasted_iota(jnp.int32, (RB, CAP), 0) + r0
        jcol = lax.broadcasted_iota(jnp.int32, (RB, CAP), 1)
        strips.append((over & (icol < jcol)).astype(jnp.float32))
    supm = jnp.concatenate(strips, axis=0)

    def _nms_cond(st):
        return st[1]

    def _nms_body(st):
        keep, _ = st
        sup = lax.dot_general(keep, supm, (((1,), (0,)), ((), ())))
        newk = jnp.where(sup > 0.5, 0.0, valid_row)
        return (newk, jnp.any(newk != keep))

    keep_row, _ = lax.while_loop(_nms_cond, _nms_body,
                                 (valid_row, jnp.bool_(True)))

    cnt_b = []
    for rb in range(CAP // RB):
        r0 = rb * RB
        icol = lax.broadcasted_iota(jnp.int32, (RB, CAP), 0) + r0
        jcol = lax.broadcasted_iota(jnp.int32, (RB, CAP), 1)
        m = keep_row * (jcol < icol).astype(jnp.float32)
        cnt_b.append(jnp.sum(m, axis=1, keepdims=True))
    pos_row = _t(jnp.concatenate(cnt_b, axis=0))

    oslot = lax.broadcasted_iota(jnp.int32, (128, CAP), 0).astype(jnp.float32)
    oh = (oslot == pos_row).astype(jnp.float32) * keep_row
    outp = jnp.concatenate([sc, cl, b0, b1, b2, b3], axis=0)
    gath = lax.dot_general(outp, oh, (((1,), (1,)), ((), ())),
                           precision=_HI)
    filled = _t(jnp.sum(oh, axis=1, keepdims=True))
    final = jnp.where(filled > 0.5, gath, -1.0)

    os_ref[...] = jnp.reshape(final[0:1, :MAX_DET], (1, 1, MAX_DET))
    oc_ref[...] = jnp.reshape(final[1:2, :MAX_DET], (1, 1, MAX_DET))
    ob_ref[...] = jnp.reshape(_t(final[2:6, :MAX_DET]), (1, MAX_DET, 4))


@jax.jit
def kernel(cls_heads, reg_heads, batch_anchors):
    B, N, C = cls_heads.shape
    nblk_a = 10
    blk_a = N // nblk_a

    scores4, classes4 = pl.pallas_call(
        _score_class_kernel,
        grid=(B, nblk_a),
        in_specs=[pl.BlockSpec((1, blk_a, C), lambda b, k: (b, k, 0))],
        out_specs=[pl.BlockSpec((1, 1, 1, blk_a), lambda b, k: (b, k, 0, 0)),
                   pl.BlockSpec((1, 1, 1, blk_a), lambda b, k: (b, k, 0, 0))],
        out_shape=[jax.ShapeDtypeStruct((B, nblk_a, 1, blk_a), jnp.float32),
                   jax.ShapeDtypeStruct((B, nblk_a, 1, blk_a), jnp.float32)],
    )(cls_heads)
    scores = jnp.reshape(scores4, (B, N))
    classes = jnp.reshape(classes4, (B, N))

    pad = NPAD - N
    s2d = jnp.reshape(jnp.pad(scores, ((0, 0), (0, pad)),
                              constant_values=-1.0), (B, ROWS, 128))
    c2d = jnp.reshape(jnp.pad(classes, ((0, 0), (0, pad))), (B, ROWS, 128))
    regT = jnp.reshape(jnp.transpose(
        jnp.pad(reg_heads, ((0, 0), (0, pad), (0, 0))), (0, 2, 1)),
        (B, 4, ROWS, 128))
    anchT = jnp.reshape(jnp.transpose(
        jnp.pad(batch_anchors, ((0, 0), (0, pad), (0, 0))), (0, 2, 1)),
        (B, 4, ROWS, 128))

    compact = pl.pallas_call(
        _compact_kernel,
        grid=(B, NBLK),
        in_specs=[pl.BlockSpec((1, ROWS, 128), lambda b, k: (b, 0, 0)),
                  pl.BlockSpec((1, BROWS, 128), lambda b, k: (b, k, 0)),
                  pl.BlockSpec((1, BROWS, 128), lambda b, k: (b, k, 0)),
                  pl.BlockSpec((1, 4, BROWS, 128), lambda b, k: (b, 0, k, 0)),
                  pl.BlockSpec((1, 4, BROWS, 128), lambda b, k: (b, 0, k, 0))],
        out_specs=pl.BlockSpec((1, 11, CAP), lambda b, k: (b, 0, 0)),
        out_shape=jax.ShapeDtypeStruct((B, 11, CAP), jnp.float32),
        scratch_shapes=[pltpu.VMEM((ROWS, 128), jnp.float32),
                        pltpu.VMEM((ROWS, 128), jnp.float32)],
    )(s2d, s2d, c2d, regT, anchT)

    out_s, out_c, out_b = pl.pallas_call(
        _finish_kernel,
        grid=(B,),
        in_specs=[pl.BlockSpec((1, 11, CAP), lambda b: (b, 0, 0))],
        out_specs=[pl.BlockSpec((1, 1, MAX_DET), lambda b: (b, 0, 0)),
                   pl.BlockSpec((1, 1, MAX_DET), lambda b: (b, 0, 0)),
                   pl.BlockSpec((1, MAX_DET, 4), lambda b: (b, 0, 0))],
        out_shape=[jax.ShapeDtypeStruct((B, 1, MAX_DET), jnp.float32),
                   jax.ShapeDtypeStruct((B, 1, MAX_DET), jnp.float32),
                   jax.ShapeDtypeStruct((B, MAX_DET, 4), jnp.float32)],
    )(compact)

    return (jnp.reshape(out_s, (B, MAX_DET)),
            jnp.reshape(out_c, (B, MAX_DET)),
            out_b)
